# Initial kernel scaffold; baseline (speedup 1.0000x reference)
import jax
import jax.numpy as jnp
from jax.experimental import pallas as pl


def kernel(loc_data, conf_data, priors, targets):
    raise NotImplementedError("write your pallas kernel here")


# speedup vs baseline: 15.1784x; 15.1784x over previous
import jax
import jax.numpy as jnp
from jax import lax
from jax.experimental import pallas as pl
from jax.experimental.pallas import tpu as pltpu

_OVERLAP_THRESH = 0.5
_NEG_POS_RATIO = 3
_VAR0, _VAR1 = 0.1, 0.2


def _row_kernel(truths_ref, priors_ref, loc_ref, conf_ref,
                out_l_ref, out_c_ref, out_n_ref):
    t = truths_ref[0]
    pT = priors_ref[...]
    NOBJ = t.shape[0]
    P = pT.shape[1]

    pcx, pcy, pw, ph = pT[0:1], pT[1:2], pT[2:3], pT[3:4]
    px0 = pcx - pw / 2.0
    py0 = pcy - ph / 2.0
    px1 = pcx + pw / 2.0
    py1 = pcy + ph / 2.0

    t_x0 = t[:, 0:1]
    t_y0 = t[:, 1:2]
    t_x1 = t[:, 2:3]
    t_y1 = t[:, 3:4]

    iw = jnp.clip(jnp.minimum(t_x1, px1) - jnp.maximum(t_x0, px0), 0.0, None)
    ih = jnp.clip(jnp.minimum(t_y1, py1) - jnp.maximum(t_y0, py0), 0.0, None)
    inter = iw * ih
    area_t = (t_x1 - t_x0) * (t_y1 - t_y0)
    area_p = (px1 - px0) * (py1 - py0)
    ov = inter / (area_t + area_p - inter)

    o_iota = lax.broadcasted_iota(jnp.int32, (NOBJ, P), 0)
    p_iota = lax.broadcasted_iota(jnp.int32, (NOBJ, P), 1)

    bto0 = jnp.max(ov, axis=0, keepdims=True)
    bti0 = jnp.min(jnp.where(ov == bto0, o_iota, NOBJ), axis=0, keepdims=True)

    maxv = jnp.max(ov, axis=1, keepdims=True)
    bpi = jnp.min(jnp.where(ov == maxv, p_iota, P), axis=1, keepdims=True)

    scat_o = jnp.max(jnp.where(p_iota == bpi, o_iota, -1), axis=0, keepdims=True)
    scat_any = scat_o >= 0
    bto = jnp.where(scat_any, 2.0, bto0)
    bti = jnp.where(scat_any, scat_o, bti0)

    pos = jnp.logical_not(bto < _OVERLAP_THRESH)
    posf = pos.astype(jnp.float32)

    onehot = (o_iota == bti).astype(jnp.float32)
    m_x0 = jnp.sum(onehot * t_x0, axis=0, keepdims=True)
    m_y0 = jnp.sum(onehot * t_y0, axis=0, keepdims=True)
    m_x1 = jnp.sum(onehot * t_x1, axis=0, keepdims=True)
    m_y1 = jnp.sum(onehot * t_y1, axis=0, keepdims=True)

    g_cx = ((m_x0 + m_x1) / 2.0 - pcx) / (_VAR0 * pw)
    g_cy = ((m_y0 + m_y1) / 2.0 - pcy) / (_VAR0 * ph)
    g_w = jnp.log((m_x1 - m_x0) / pw) / _VAR1
    g_h = jnp.log((m_y1 - m_y0) / ph) / _VAR1

    lT = loc_ref[0]

    def smooth_l1(d):
        ad = jnp.abs(d)
        return jnp.where(ad < 1.0, 0.5 * d * d, ad - 0.5)

    sl = (smooth_l1(lT[0:1] - g_cx) + smooth_l1(lT[1:2] - g_cy)
          + smooth_l1(lT[2:3] - g_w) + smooth_l1(lT[3:4] - g_h))
    loss_l = jnp.sum(sl * posf)

    cT = conf_ref[0]
    c0, c1 = cT[0:1], cT[1:2]
    cmx = jnp.maximum(c0, c1)
    lse = jnp.log(jnp.exp(c0 - cmx) + jnp.exp(c1 - cmx)) + cmx
    ce = lse - jnp.where(pos, c1, c0)

    mneg = jnp.where(pos, 0.0, ce)
    npos = jnp.sum(pos.astype(jnp.int32))
    num_neg = jnp.minimum(_NEG_POS_RATIO * npos, P - 1)

    mbits = lax.bitcast_convert_type(mneg, jnp.int32)
    res = jnp.int32(0)
    for bit in range(30, -1, -1):
        cand = res | jnp.int32(1 << bit)
        cnt = jnp.sum((mbits >= cand).astype(jnp.int32))
        res = jnp.where(cnt >= num_neg, cand, res)

    gt_mask = mbits > res
    sum_gt = jnp.sum(jnp.where(gt_mask, mneg, 0.0))
    c_gt = jnp.sum(gt_mask.astype(jnp.int32))
    eq_mask = mbits == res
    tie_sum = jnp.sum(jnp.where(eq_mask, mneg, 0.0))
    tie_cnt = jnp.sum(eq_mask.astype(jnp.int32))
    neg_contrib = sum_gt + tie_sum * (
        (num_neg - c_gt).astype(jnp.float32) / tie_cnt.astype(jnp.float32))

    loss_c = jnp.sum(ce * posf) + neg_contrib

    out_l_ref[...] = jnp.reshape(loss_l, (1, 1, 1))
    out_c_ref[...] = jnp.reshape(loss_c, (1, 1, 1))
    out_n_ref[...] = jnp.reshape(npos.astype(jnp.float32), (1, 1, 1))


def _reduce_kernel(lr_ref, cr_ref, nr_ref, out_l_ref, out_c_ref):
    n = jnp.sum(nr_ref[...])
    out_l_ref[...] = jnp.reshape(jnp.sum(lr_ref[...]) / n, (1, 1))
    out_c_ref[...] = jnp.reshape(jnp.sum(cr_ref[...]) / n, (1, 1))


def kernel(loc_data, conf_data, priors, targets):
    B, P, _ = loc_data.shape
    NOBJ = targets.shape[1]

    loc_T = jnp.transpose(loc_data, (0, 2, 1))
    conf_T = jnp.transpose(conf_data, (0, 2, 1))
    priors_T = priors[:P].T
    truths = targets[:, :, :4]

    f32 = jnp.float32
    lr, cr, nr = pl.pallas_call(
        _row_kernel,
        grid=(B,),
        in_specs=[
            pl.BlockSpec((1, NOBJ, 4), lambda b: (b, 0, 0)),
            pl.BlockSpec((4, P), lambda b: (0, 0)),
            pl.BlockSpec((1, 4, P), lambda b: (b, 0, 0)),
            pl.BlockSpec((1, 2, P), lambda b: (b, 0, 0)),
        ],
        out_specs=[
            pl.BlockSpec((1, 1, 1), lambda b: (b, 0, 0)),
            pl.BlockSpec((1, 1, 1), lambda b: (b, 0, 0)),
            pl.BlockSpec((1, 1, 1), lambda b: (b, 0, 0)),
        ],
        out_shape=[
            jax.ShapeDtypeStruct((B, 1, 1), f32),
            jax.ShapeDtypeStruct((B, 1, 1), f32),
            jax.ShapeDtypeStruct((B, 1, 1), f32),
        ],
        compiler_params=pltpu.CompilerParams(
            dimension_semantics=("parallel",)),
    )(truths, priors_T, loc_T, conf_T)

    loss_l, loss_c = pl.pallas_call(
        _reduce_kernel,
        out_shape=[
            jax.ShapeDtypeStruct((1, 1), f32),
            jax.ShapeDtypeStruct((1, 1), f32),
        ],
    )(lr, cr, nr)

    return loss_l[0, 0], loss_c[0, 0]


# speedup vs baseline: 25.5856x; 1.6857x over previous
import jax
import jax.numpy as jnp
from jax import lax
from jax.experimental import pallas as pl
from jax.experimental.pallas import tpu as pltpu

_OVERLAP_THRESH = 0.5
_NEG_POS_RATIO = 3
_VAR0, _VAR1 = 0.1, 0.2

_SUB = 8


def _row_kernel(truths_ref, truths_t_ref, priors_ref, loc_ref, conf_ref,
                out_l_ref, out_c_ref, out_n_ref):
    t = truths_ref[0]
    tT = truths_t_ref[0]
    pR = priors_ref[...]
    NOBJ = t.shape[0]
    SUB, LN = pR.shape[1], pR.shape[2]
    P = SUB * LN

    pcx, pcy, pw, ph = pR[0], pR[1], pR[2], pR[3]
    px0 = pcx - pw / 2.0
    py0 = pcy - ph / 2.0
    px1 = pcx + pw / 2.0
    py1 = pcy + ph / 2.0
    area_p = (px1 - px0) * (py1 - py0)

    t_x0 = jnp.reshape(t[:, 0:1], (NOBJ, 1, 1))
    t_y0 = jnp.reshape(t[:, 1:2], (NOBJ, 1, 1))
    t_x1 = jnp.reshape(t[:, 2:3], (NOBJ, 1, 1))
    t_y1 = jnp.reshape(t[:, 3:4], (NOBJ, 1, 1))
    area_t = (t_x1 - t_x0) * (t_y1 - t_y0)

    iw = jnp.maximum(jnp.minimum(t_x1, px1[None]) - jnp.maximum(t_x0, px0[None]), 0.0)
    ih = jnp.maximum(jnp.minimum(t_y1, py1[None]) - jnp.maximum(t_y0, py0[None]), 0.0)
    inter = iw * ih
    ov = inter / ((area_t + area_p[None]) - inter)

    o_iota = lax.broadcasted_iota(jnp.int32, (NOBJ, SUB, LN), 0)
    p_idx = (lax.broadcasted_iota(jnp.int32, (SUB, LN), 0) * LN
             + lax.broadcasted_iota(jnp.int32, (SUB, LN), 1))

    bto0 = jnp.max(ov, axis=0)
    bti0 = jnp.min(jnp.where(ov == bto0[None], o_iota, NOBJ), axis=0)

    maxv = jnp.max(jnp.max(ov, axis=2, keepdims=True), axis=1, keepdims=True)
    cand = jnp.where(ov == maxv, p_idx[None], P)
    bpi = jnp.min(jnp.min(cand, axis=2, keepdims=True), axis=1, keepdims=True)

    scat_o = jnp.max(jnp.where(p_idx[None] == bpi, o_iota, -1), axis=0)
    scat_any = scat_o >= 0
    bto = jnp.where(scat_any, 2.0, bto0)
    bti = jnp.where(scat_any, scat_o, bti0)

    pos = jnp.logical_not(bto < _OVERLAP_THRESH)
    posf = pos.astype(jnp.float32)

    onehot = (o_iota == bti[None]).astype(jnp.float32)
    matched = lax.dot_general(tT, onehot,
                              dimension_numbers=(((1,), (0,)), ((), ())),
                              preferred_element_type=jnp.float32)
    m_x0, m_y0, m_x1, m_y1 = matched[0], matched[1], matched[2], matched[3]

    g_cx = ((m_x0 + m_x1) / 2.0 - pcx) / (_VAR0 * pw)
    g_cy = ((m_y0 + m_y1) / 2.0 - pcy) / (_VAR0 * ph)
    g_w = jnp.log((m_x1 - m_x0) / pw) / _VAR1
    g_h = jnp.log((m_y1 - m_y0) / ph) / _VAR1

    lR = loc_ref[0]

    def smooth_l1(d):
        ad = jnp.abs(d)
        return jnp.where(ad < 1.0, 0.5 * d * d, ad - 0.5)

    sl = (smooth_l1(lR[0] - g_cx) + smooth_l1(lR[1] - g_cy)
          + smooth_l1(lR[2] - g_w) + smooth_l1(lR[3] - g_h))
    loss_l = jnp.sum(sl * posf)

    cR = conf_ref[0]
    c0, c1 = cR[0], cR[1]
    cmx = jnp.maximum(c0, c1)
    lse = jnp.log(jnp.exp(c0 - cmx) + jnp.exp(c1 - cmx)) + cmx
    ce = lse - jnp.where(pos, c1, c0)

    mneg = jnp.where(pos, 0.0, ce)
    npos = jnp.sum(pos.astype(jnp.int32))
    num_neg = jnp.minimum(_NEG_POS_RATIO * npos, P - 1)

    mbits = lax.bitcast_convert_type(mneg, jnp.int32)
    res = jnp.int32(0)
    for bit in range(30, -1, -1):
        cand_t = res | jnp.int32(1 << bit)
        cnt = jnp.sum((mbits >= cand_t).astype(jnp.int32))
        res = jnp.where(cnt >= num_neg, cand_t, res)

    gt_mask = mbits > res
    sum_gt = jnp.sum(jnp.where(gt_mask, mneg, 0.0))
    c_gt = jnp.sum(gt_mask.astype(jnp.int32))
    eq_mask = mbits == res
    tie_sum = jnp.sum(jnp.where(eq_mask, mneg, 0.0))
    tie_cnt = jnp.sum(eq_mask.astype(jnp.int32))
    neg_contrib = sum_gt + tie_sum * (
        (num_neg - c_gt).astype(jnp.float32) / tie_cnt.astype(jnp.float32))

    loss_c = jnp.sum(ce * posf) + neg_contrib

    out_l_ref[...] = jnp.reshape(loss_l, (1, 1, 1))
    out_c_ref[...] = jnp.reshape(loss_c, (1, 1, 1))
    out_n_ref[...] = jnp.reshape(npos.astype(jnp.float32), (1, 1, 1))


def _reduce_kernel(lr_ref, cr_ref, nr_ref, out_l_ref, out_c_ref):
    n = jnp.sum(nr_ref[...])
    out_l_ref[...] = jnp.reshape(jnp.sum(lr_ref[...]) / n, (1, 1))
    out_c_ref[...] = jnp.reshape(jnp.sum(cr_ref[...]) / n, (1, 1))


def kernel(loc_data, conf_data, priors, targets):
    B, P, _ = loc_data.shape
    NOBJ = targets.shape[1]
    SUB = _SUB
    LN = P // SUB

    loc_R = jnp.transpose(loc_data, (0, 2, 1)).reshape(B, 4, SUB, LN)
    conf_R = jnp.transpose(conf_data, (0, 2, 1)).reshape(B, 2, SUB, LN)
    priors_R = priors[:P].T.reshape(4, SUB, LN)
    truths = targets[:, :, :4]
    truths_T = jnp.transpose(truths, (0, 2, 1))

    f32 = jnp.float32
    lr, cr, nr = pl.pallas_call(
        _row_kernel,
        grid=(B,),
        in_specs=[
            pl.BlockSpec((1, NOBJ, 4), lambda b: (b, 0, 0)),
            pl.BlockSpec((1, 4, NOBJ), lambda b: (b, 0, 0)),
            pl.BlockSpec((4, SUB, LN), lambda b: (0, 0, 0)),
            pl.BlockSpec((1, 4, SUB, LN), lambda b: (b, 0, 0, 0)),
            pl.BlockSpec((1, 2, SUB, LN), lambda b: (b, 0, 0, 0)),
        ],
        out_specs=[
            pl.BlockSpec((1, 1, 1), lambda b: (b, 0, 0)),
            pl.BlockSpec((1, 1, 1), lambda b: (b, 0, 0)),
            pl.BlockSpec((1, 1, 1), lambda b: (b, 0, 0)),
        ],
        out_shape=[
            jax.ShapeDtypeStruct((B, 1, 1), f32),
            jax.ShapeDtypeStruct((B, 1, 1), f32),
            jax.ShapeDtypeStruct((B, 1, 1), f32),
        ],
        compiler_params=pltpu.CompilerParams(
            dimension_semantics=("parallel",)),
    )(truths, truths_T, priors_R, loc_R, conf_R)

    loss_l, loss_c = pl.pallas_call(
        _reduce_kernel,
        out_shape=[
            jax.ShapeDtypeStruct((1, 1), f32),
            jax.ShapeDtypeStruct((1, 1), f32),
        ],
    )(lr, cr, nr)

    return loss_l[0, 0], loss_c[0, 0]


# speedup vs baseline: 40.7714x; 1.5935x over previous
import jax
import jax.numpy as jnp
from jax import lax
from jax.experimental import pallas as pl
from jax.experimental.pallas import tpu as pltpu

_OVERLAP_THRESH = 0.5
_NEG_POS_RATIO = 3
_VAR0, _VAR1 = 0.1, 0.2

_SUB = 8


def _row_kernel(truths_ref, truths_t_ref, priors_ref, loc_ref, conf_ref,
                mneg_ref, out_l_ref, out_cp_ref, out_n_ref):
    t = truths_ref[0]
    tT = truths_t_ref[0]
    pR = priors_ref[...]
    NOBJ = t.shape[0]
    SUB, LN = pR.shape[1], pR.shape[2]
    P = SUB * LN

    pcx, pcy, pw, ph = pR[0], pR[1], pR[2], pR[3]
    px0 = pcx - pw / 2.0
    py0 = pcy - ph / 2.0
    px1 = pcx + pw / 2.0
    py1 = pcy + ph / 2.0
    area_p = (px1 - px0) * (py1 - py0)

    t_x0 = jnp.reshape(t[:, 0:1], (NOBJ, 1, 1))
    t_y0 = jnp.reshape(t[:, 1:2], (NOBJ, 1, 1))
    t_x1 = jnp.reshape(t[:, 2:3], (NOBJ, 1, 1))
    t_y1 = jnp.reshape(t[:, 3:4], (NOBJ, 1, 1))
    area_t = (t_x1 - t_x0) * (t_y1 - t_y0)

    iw = jnp.maximum(jnp.minimum(t_x1, px1[None]) - jnp.maximum(t_x0, px0[None]), 0.0)
    ih = jnp.maximum(jnp.minimum(t_y1, py1[None]) - jnp.maximum(t_y0, py0[None]), 0.0)
    inter = iw * ih
    ov = inter / ((area_t + area_p[None]) - inter)

    o_iota = lax.broadcasted_iota(jnp.int32, (NOBJ, SUB, LN), 0)
    p_idx = (lax.broadcasted_iota(jnp.int32, (SUB, LN), 0) * LN
             + lax.broadcasted_iota(jnp.int32, (SUB, LN), 1))

    bto0 = jnp.max(ov, axis=0)
    bti0 = jnp.min(jnp.where(ov == bto0[None], o_iota, NOBJ), axis=0)

    maxv = jnp.max(jnp.max(ov, axis=2, keepdims=True), axis=1, keepdims=True)
    cand = jnp.where(ov == maxv, p_idx[None], P)
    bpi = jnp.min(jnp.min(cand, axis=2, keepdims=True), axis=1, keepdims=True)

    scat_o = jnp.max(jnp.where(p_idx[None] == bpi, o_iota, -1), axis=0)
    scat_any = scat_o >= 0
    bto = jnp.where(scat_any, 2.0, bto0)
    bti = jnp.where(scat_any, scat_o, bti0)

    pos = jnp.logical_not(bto < _OVERLAP_THRESH)
    posf = pos.astype(jnp.float32)

    onehot = (o_iota == bti[None]).astype(jnp.float32)
    matched = lax.dot_general(tT, onehot,
                              dimension_numbers=(((1,), (0,)), ((), ())),
                              preferred_element_type=jnp.float32)
    m_x0, m_y0, m_x1, m_y1 = matched[0], matched[1], matched[2], matched[3]

    g_cx = ((m_x0 + m_x1) / 2.0 - pcx) / (_VAR0 * pw)
    g_cy = ((m_y0 + m_y1) / 2.0 - pcy) / (_VAR0 * ph)
    g_w = jnp.log((m_x1 - m_x0) / pw) / _VAR1
    g_h = jnp.log((m_y1 - m_y0) / ph) / _VAR1

    lR = loc_ref[0]

    def smooth_l1(d):
        ad = jnp.abs(d)
        return jnp.where(ad < 1.0, 0.5 * d * d, ad - 0.5)

    sl = (smooth_l1(lR[0] - g_cx) + smooth_l1(lR[1] - g_cy)
          + smooth_l1(lR[2] - g_w) + smooth_l1(lR[3] - g_h))
    loss_l = jnp.sum(sl * posf)

    cR = conf_ref[0]
    c0, c1 = cR[0], cR[1]
    cmx = jnp.maximum(c0, c1)
    lse = jnp.log(jnp.exp(c0 - cmx) + jnp.exp(c1 - cmx)) + cmx
    ce = lse - jnp.where(pos, c1, c0)

    mneg_ref[0] = jnp.where(pos, 0.0, ce)
    npos = jnp.sum(pos.astype(jnp.int32))

    out_l_ref[...] = jnp.reshape(loss_l, (1, 1, 1))
    out_cp_ref[...] = jnp.reshape(jnp.sum(ce * posf), (1, 1, 1))
    out_n_ref[...] = jnp.reshape(npos, (1, 1, 1))


def _mine_kernel(mneg_ref, lossl_ref, cepos_ref, npos_ref,
                 out_l_ref, out_c_ref):
    mneg = mneg_ref[...]
    B, SUB, LN = mneg.shape
    P = SUB * LN
    npos = npos_ref[...]
    num_neg = jnp.minimum(_NEG_POS_RATIO * npos, P - 1)

    mbits = lax.bitcast_convert_type(mneg, jnp.int32)
    res = jnp.zeros((B, 1, 1), jnp.int32)
    for bit in range(30, -1, -1):
        cand = res | jnp.int32(1 << bit)
        cnt = jnp.sum((mbits >= cand).astype(jnp.int32), axis=2, keepdims=True)
        cnt = jnp.sum(cnt, axis=1, keepdims=True)
        res = jnp.where(cnt >= num_neg, cand, res)

    gt_mask = mbits > res
    sum_gt = jnp.sum(jnp.sum(jnp.where(gt_mask, mneg, 0.0),
                             axis=2, keepdims=True), axis=1, keepdims=True)
    c_gt = jnp.sum(jnp.sum(gt_mask.astype(jnp.int32),
                           axis=2, keepdims=True), axis=1, keepdims=True)
    eq_mask = mbits == res
    tie_sum = jnp.sum(jnp.sum(jnp.where(eq_mask, mneg, 0.0),
                              axis=2, keepdims=True), axis=1, keepdims=True)
    tie_cnt = jnp.sum(jnp.sum(eq_mask.astype(jnp.int32),
                              axis=2, keepdims=True), axis=1, keepdims=True)
    neg_contrib = sum_gt + tie_sum * (
        (num_neg - c_gt).astype(jnp.float32) / tie_cnt.astype(jnp.float32))

    loss_c_rows = cepos_ref[...] + neg_contrib
    n = jnp.sum(npos).astype(jnp.float32)
    out_l_ref[...] = jnp.reshape(jnp.sum(lossl_ref[...]) / n, (1, 1))
    out_c_ref[...] = jnp.reshape(jnp.sum(loss_c_rows) / n, (1, 1))


def kernel(loc_data, conf_data, priors, targets):
    B, P, _ = loc_data.shape
    NOBJ = targets.shape[1]
    SUB = _SUB
    LN = P // SUB

    loc_R = jnp.transpose(loc_data, (0, 2, 1)).reshape(B, 4, SUB, LN)
    conf_R = jnp.transpose(conf_data, (0, 2, 1)).reshape(B, 2, SUB, LN)
    priors_R = priors[:P].T.reshape(4, SUB, LN)
    truths = targets[:, :, :4]
    truths_T = jnp.transpose(truths, (0, 2, 1))

    f32 = jnp.float32
    mneg, lossl, cepos, npos = pl.pallas_call(
        _row_kernel,
        grid=(B,),
        in_specs=[
            pl.BlockSpec((1, NOBJ, 4), lambda b: (b, 0, 0)),
            pl.BlockSpec((1, 4, NOBJ), lambda b: (b, 0, 0)),
            pl.BlockSpec((4, SUB, LN), lambda b: (0, 0, 0)),
            pl.BlockSpec((1, 4, SUB, LN), lambda b: (b, 0, 0, 0)),
            pl.BlockSpec((1, 2, SUB, LN), lambda b: (b, 0, 0, 0)),
        ],
        out_specs=[
            pl.BlockSpec((1, SUB, LN), lambda b: (b, 0, 0)),
            pl.BlockSpec((1, 1, 1), lambda b: (b, 0, 0)),
            pl.BlockSpec((1, 1, 1), lambda b: (b, 0, 0)),
            pl.BlockSpec((1, 1, 1), lambda b: (b, 0, 0)),
        ],
        out_shape=[
            jax.ShapeDtypeStruct((B, SUB, LN), f32),
            jax.ShapeDtypeStruct((B, 1, 1), f32),
            jax.ShapeDtypeStruct((B, 1, 1), f32),
            jax.ShapeDtypeStruct((B, 1, 1), jnp.int32),
        ],
        compiler_params=pltpu.CompilerParams(
            dimension_semantics=("parallel",)),
    )(truths, truths_T, priors_R, loc_R, conf_R)

    loss_l, loss_c = pl.pallas_call(
        _mine_kernel,
        out_shape=[
            jax.ShapeDtypeStruct((1, 1), f32),
            jax.ShapeDtypeStruct((1, 1), f32),
        ],
    )(mneg, lossl, cepos, npos)

    return loss_l[0, 0], loss_c[0, 0]


# speedup vs baseline: 40.8887x; 1.0029x over previous
import jax
import jax.numpy as jnp
from jax import lax
from jax.experimental import pallas as pl
from jax.experimental.pallas import tpu as pltpu

_OVERLAP_THRESH = 0.5
_NEG_POS_RATIO = 3
_VAR0, _VAR1 = 0.1, 0.2

_SUB = 8


def _row_kernel(truths_ref, truths_t_ref, priors_ref, loc_ref, conf_ref,
                mneg_ref, out_l_ref, out_cp_ref, out_n_ref):
    t = truths_ref[0]
    tT = truths_t_ref[0]
    pR = priors_ref[...]
    NOBJ = t.shape[0]
    SUB, LN = pR.shape[1], pR.shape[2]
    P = SUB * LN

    pcx, pcy, pw, ph = pR[0], pR[1], pR[2], pR[3]
    px0 = pcx - pw / 2.0
    py0 = pcy - ph / 2.0
    px1 = pcx + pw / 2.0
    py1 = pcy + ph / 2.0
    area_p = (px1 - px0) * (py1 - py0)

    t_x0 = jnp.reshape(t[:, 0:1], (NOBJ, 1, 1))
    t_y0 = jnp.reshape(t[:, 1:2], (NOBJ, 1, 1))
    t_x1 = jnp.reshape(t[:, 2:3], (NOBJ, 1, 1))
    t_y1 = jnp.reshape(t[:, 3:4], (NOBJ, 1, 1))
    area_t = (t_x1 - t_x0) * (t_y1 - t_y0)

    iw = jnp.maximum(jnp.minimum(t_x1, px1[None]) - jnp.maximum(t_x0, px0[None]), 0.0)
    ih = jnp.maximum(jnp.minimum(t_y1, py1[None]) - jnp.maximum(t_y0, py0[None]), 0.0)
    inter = iw * ih
    ov = inter / ((area_t + area_p[None]) - inter)

    o_iota = lax.broadcasted_iota(jnp.int32, (NOBJ, SUB, LN), 0)
    p_idx = (lax.broadcasted_iota(jnp.int32, (SUB, LN), 0) * LN
             + lax.broadcasted_iota(jnp.int32, (SUB, LN), 1))

    bto0 = jnp.max(ov, axis=0)
    bti0 = jnp.min(jnp.where(ov == bto0[None], o_iota, NOBJ), axis=0)

    maxv = jnp.max(jnp.max(ov, axis=2, keepdims=True), axis=1, keepdims=True)
    cand = jnp.where(ov == maxv, p_idx[None], P)
    bpi = jnp.min(jnp.min(cand, axis=2, keepdims=True), axis=1, keepdims=True)

    scat_o = jnp.max(jnp.where(p_idx[None] == bpi, o_iota, -1), axis=0)
    scat_any = scat_o >= 0
    bti = jnp.where(scat_any, scat_o, bti0)

    pos = scat_any | jnp.logical_not(bto0 < _OVERLAP_THRESH)
    posf = pos.astype(jnp.float32)

    onehot = (o_iota == bti[None]).astype(jnp.float32)
    matched = lax.dot_general(tT, onehot,
                              dimension_numbers=(((1,), (0,)), ((), ())),
                              preferred_element_type=jnp.float32)
    m_x0, m_y0, m_x1, m_y1 = matched[0], matched[1], matched[2], matched[3]

    g_cx = ((m_x0 + m_x1) / 2.0 - pcx) / (_VAR0 * pw)
    g_cy = ((m_y0 + m_y1) / 2.0 - pcy) / (_VAR0 * ph)
    g_w = jnp.log((m_x1 - m_x0) / pw) / _VAR1
    g_h = jnp.log((m_y1 - m_y0) / ph) / _VAR1

    lR = loc_ref[0]

    def smooth_l1(d):
        ad = jnp.abs(d)
        return jnp.where(ad < 1.0, 0.5 * d * d, ad - 0.5)

    sl = (smooth_l1(lR[0] - g_cx) + smooth_l1(lR[1] - g_cy)
          + smooth_l1(lR[2] - g_w) + smooth_l1(lR[3] - g_h))
    loss_l = jnp.sum(sl * posf)

    cR = conf_ref[0]
    c0, c1 = cR[0], cR[1]
    cmx = jnp.maximum(c0, c1)
    lse = jnp.log(jnp.exp(c0 - cmx) + jnp.exp(c1 - cmx)) + cmx
    ce = lse - jnp.where(pos, c1, c0)

    mneg_ref[0] = jnp.where(pos, 0.0, ce)
    npos = jnp.sum(pos.astype(jnp.int32))

    out_l_ref[...] = jnp.reshape(loss_l, (1, 1, 1))
    out_cp_ref[...] = jnp.reshape(jnp.sum(ce * posf), (1, 1, 1))
    out_n_ref[...] = jnp.reshape(npos, (1, 1, 1))


def _mine_kernel(mneg_ref, lossl_ref, cepos_ref, npos_ref,
                 out_l_ref, out_c_ref):
    mneg = mneg_ref[...]
    B, SUB, LN = mneg.shape
    P = SUB * LN
    npos = npos_ref[...]
    num_neg = jnp.minimum(_NEG_POS_RATIO * npos, P - 1)

    mbits = lax.bitcast_convert_type(mneg, jnp.int32)
    res = jnp.zeros((B, 1, 1), jnp.int32)
    for bit in range(30, -1, -1):
        cand = res | jnp.int32(1 << bit)
        cnt = jnp.sum((mbits >= cand).astype(jnp.int32), axis=2, keepdims=True)
        cnt = jnp.sum(cnt, axis=1, keepdims=True)
        res = jnp.where(cnt >= num_neg, cand, res)

    t_val = lax.bitcast_convert_type(res, jnp.float32)
    gt_mask = mbits > res
    sum_gt = jnp.sum(jnp.sum(jnp.where(gt_mask, mneg, 0.0),
                             axis=2, keepdims=True), axis=1, keepdims=True)
    c_gt = jnp.sum(jnp.sum(gt_mask.astype(jnp.int32),
                           axis=2, keepdims=True), axis=1, keepdims=True)
    neg_contrib = sum_gt + (num_neg - c_gt).astype(jnp.float32) * t_val

    loss_c_rows = cepos_ref[...] + neg_contrib
    n = jnp.sum(npos).astype(jnp.float32)
    out_l_ref[...] = jnp.reshape(jnp.sum(lossl_ref[...]) / n, (1, 1))
    out_c_ref[...] = jnp.reshape(jnp.sum(loss_c_rows) / n, (1, 1))


def kernel(loc_data, conf_data, priors, targets):
    B, P, _ = loc_data.shape
    NOBJ = targets.shape[1]
    SUB = _SUB
    LN = P // SUB

    loc_R = jnp.transpose(loc_data, (0, 2, 1)).reshape(B, 4, SUB, LN)
    conf_R = jnp.transpose(conf_data, (0, 2, 1)).reshape(B, 2, SUB, LN)
    priors_R = priors[:P].T.reshape(4, SUB, LN)
    truths = targets[:, :, :4]
    truths_T = jnp.transpose(truths, (0, 2, 1))

    f32 = jnp.float32
    mneg, lossl, cepos, npos = pl.pallas_call(
        _row_kernel,
        grid=(B,),
        in_specs=[
            pl.BlockSpec((1, NOBJ, 4), lambda b: (b, 0, 0)),
            pl.BlockSpec((1, 4, NOBJ), lambda b: (b, 0, 0)),
            pl.BlockSpec((4, SUB, LN), lambda b: (0, 0, 0)),
            pl.BlockSpec((1, 4, SUB, LN), lambda b: (b, 0, 0, 0)),
            pl.BlockSpec((1, 2, SUB, LN), lambda b: (b, 0, 0, 0)),
        ],
        out_specs=[
            pl.BlockSpec((1, SUB, LN), lambda b: (b, 0, 0)),
            pl.BlockSpec((1, 1, 1), lambda b: (b, 0, 0)),
            pl.BlockSpec((1, 1, 1), lambda b: (b, 0, 0)),
            pl.BlockSpec((1, 1, 1), lambda b: (b, 0, 0)),
        ],
        out_shape=[
            jax.ShapeDtypeStruct((B, SUB, LN), f32),
            jax.ShapeDtypeStruct((B, 1, 1), f32),
            jax.ShapeDtypeStruct((B, 1, 1), f32),
            jax.ShapeDtypeStruct((B, 1, 1), jnp.int32),
        ],
        compiler_params=pltpu.CompilerParams(
            dimension_semantics=("parallel",)),
    )(truths, truths_T, priors_R, loc_R, conf_R)

    loss_l, loss_c = pl.pallas_call(
        _mine_kernel,
        out_shape=[
            jax.ShapeDtypeStruct((1, 1), f32),
            jax.ShapeDtypeStruct((1, 1), f32),
        ],
    )(mneg, lossl, cepos, npos)

    return loss_l[0, 0], loss_c[0, 0]


# speedup vs baseline: 41.5450x; 1.0161x over previous
import jax
import jax.numpy as jnp
from jax import lax
from jax.experimental import pallas as pl
from jax.experimental.pallas import tpu as pltpu

_OVERLAP_THRESH = 0.5
_NEG_POS_RATIO = 3
_VAR0, _VAR1 = 0.1, 0.2

_SUB = 8


def _row_kernel(truths_ref, truths_t_ref, priors_ref, loc_ref, conf_ref,
                out_l_ref, out_c_ref,
                mneg_ref, part_ref):
    b = pl.program_id(0)
    B = pl.num_programs(0)
    t = truths_ref[0]
    tT = truths_t_ref[0]
    pR = priors_ref[...]
    NOBJ = t.shape[0]
    SUB, LN = pR.shape[1], pR.shape[2]
    P = SUB * LN

    pcx, pcy, pw, ph = pR[0], pR[1], pR[2], pR[3]
    px0 = pcx - pw / 2.0
    py0 = pcy - ph / 2.0
    px1 = pcx + pw / 2.0
    py1 = pcy + ph / 2.0
    area_p = (px1 - px0) * (py1 - py0)

    t_x0 = jnp.reshape(t[:, 0:1], (NOBJ, 1, 1))
    t_y0 = jnp.reshape(t[:, 1:2], (NOBJ, 1, 1))
    t_x1 = jnp.reshape(t[:, 2:3], (NOBJ, 1, 1))
    t_y1 = jnp.reshape(t[:, 3:4], (NOBJ, 1, 1))
    area_t = (t_x1 - t_x0) * (t_y1 - t_y0)

    iw = jnp.maximum(jnp.minimum(t_x1, px1[None]) - jnp.maximum(t_x0, px0[None]), 0.0)
    ih = jnp.maximum(jnp.minimum(t_y1, py1[None]) - jnp.maximum(t_y0, py0[None]), 0.0)
    inter = iw * ih
    ov = inter / ((area_t + area_p[None]) - inter)

    o_iota = lax.broadcasted_iota(jnp.int32, (NOBJ, SUB, LN), 0)
    p_idx = (lax.broadcasted_iota(jnp.int32, (SUB, LN), 0) * LN
             + lax.broadcasted_iota(jnp.int32, (SUB, LN), 1))

    bto0 = jnp.max(ov, axis=0)
    bti0 = jnp.min(jnp.where(ov == bto0[None], o_iota, NOBJ), axis=0)

    maxv = jnp.max(jnp.max(ov, axis=2, keepdims=True), axis=1, keepdims=True)
    cand = jnp.where(ov == maxv, p_idx[None], P)
    bpi = jnp.min(jnp.min(cand, axis=2, keepdims=True), axis=1, keepdims=True)

    scat_o = jnp.max(jnp.where(p_idx[None] == bpi, o_iota, -1), axis=0)
    scat_any = scat_o >= 0
    bti = jnp.where(scat_any, scat_o, bti0)

    pos = scat_any | jnp.logical_not(bto0 < _OVERLAP_THRESH)
    posf = pos.astype(jnp.float32)

    onehot = (o_iota == bti[None]).astype(jnp.float32)
    matched = lax.dot_general(tT, onehot,
                              dimension_numbers=(((1,), (0,)), ((), ())),
                              preferred_element_type=jnp.float32)
    m_x0, m_y0, m_x1, m_y1 = matched[0], matched[1], matched[2], matched[3]

    g_cx = ((m_x0 + m_x1) / 2.0 - pcx) / (_VAR0 * pw)
    g_cy = ((m_y0 + m_y1) / 2.0 - pcy) / (_VAR0 * ph)
    g_w = jnp.log((m_x1 - m_x0) / pw) / _VAR1
    g_h = jnp.log((m_y1 - m_y0) / ph) / _VAR1

    lR = loc_ref[0]

    def smooth_l1(d):
        ad = jnp.abs(d)
        return jnp.where(ad < 1.0, 0.5 * d * d, ad - 0.5)

    sl = (smooth_l1(lR[0] - g_cx) + smooth_l1(lR[1] - g_cy)
          + smooth_l1(lR[2] - g_w) + smooth_l1(lR[3] - g_h))
    loss_l = jnp.sum(sl * posf)

    cR = conf_ref[0]
    c0, c1 = cR[0], cR[1]
    cmx = jnp.maximum(c0, c1)
    lse = jnp.log(jnp.exp(c0 - cmx) + jnp.exp(c1 - cmx)) + cmx
    ce = lse - jnp.where(pos, c1, c0)

    mneg_ref[b] = jnp.where(pos, 0.0, ce)
    npos = jnp.sum(posf)
    part_ref[b] = jnp.stack(
        [jnp.reshape(loss_l, (1,)), jnp.reshape(jnp.sum(ce * posf), (1,)),
         jnp.reshape(npos, (1,))], axis=0)

    @pl.when(b == B - 1)
    def _mine():
        mneg = mneg_ref[...]
        parts = part_ref[...]
        nposv = parts[:, 2:3, :]
        num_neg = jnp.minimum(
            jnp.float32(_NEG_POS_RATIO) * nposv, jnp.float32(P - 1))

        mbits = lax.bitcast_convert_type(mneg, jnp.int32)
        res = jnp.zeros((B, 1, 1), jnp.int32)
        for bit in range(30, -1, -1):
            cand_t = res | jnp.int32(1 << bit)
            cnt = jnp.sum((mbits >= cand_t).astype(jnp.float32),
                          axis=2, keepdims=True)
            cnt = jnp.sum(cnt, axis=1, keepdims=True)
            res = jnp.where(cnt >= num_neg, cand_t, res)

        t_val = lax.bitcast_convert_type(res, jnp.float32)
        gt_mask = mbits > res
        sum_gt = jnp.sum(jnp.sum(jnp.where(gt_mask, mneg, 0.0),
                                 axis=2, keepdims=True), axis=1, keepdims=True)
        c_gt = jnp.sum(jnp.sum(gt_mask.astype(jnp.float32),
                               axis=2, keepdims=True), axis=1, keepdims=True)
        neg_contrib = sum_gt + (num_neg - c_gt) * t_val

        loss_c_rows = parts[:, 1:2, :] + neg_contrib
        n = jnp.sum(nposv)
        out_l_ref[...] = jnp.reshape(jnp.sum(parts[:, 0:1, :]) / n, (1, 1))
        out_c_ref[...] = jnp.reshape(jnp.sum(loss_c_rows) / n, (1, 1))


def kernel(loc_data, conf_data, priors, targets):
    B, P, _ = loc_data.shape
    NOBJ = targets.shape[1]
    SUB = _SUB
    LN = P // SUB

    loc_R = jnp.transpose(loc_data, (0, 2, 1)).reshape(B, 4, SUB, LN)
    conf_R = jnp.transpose(conf_data, (0, 2, 1)).reshape(B, 2, SUB, LN)
    priors_R = priors[:P].T.reshape(4, SUB, LN)
    truths = targets[:, :, :4]
    truths_T = jnp.transpose(truths, (0, 2, 1))

    f32 = jnp.float32
    loss_l, loss_c = pl.pallas_call(
        _row_kernel,
        grid=(B,),
        in_specs=[
            pl.BlockSpec((1, NOBJ, 4), lambda b: (b, 0, 0)),
            pl.BlockSpec((1, 4, NOBJ), lambda b: (b, 0, 0)),
            pl.BlockSpec((4, SUB, LN), lambda b: (0, 0, 0)),
            pl.BlockSpec((1, 4, SUB, LN), lambda b: (b, 0, 0, 0)),
            pl.BlockSpec((1, 2, SUB, LN), lambda b: (b, 0, 0, 0)),
        ],
        out_specs=[
            pl.BlockSpec((1, 1), lambda b: (0, 0)),
            pl.BlockSpec((1, 1), lambda b: (0, 0)),
        ],
        out_shape=[
            jax.ShapeDtypeStruct((1, 1), f32),
            jax.ShapeDtypeStruct((1, 1), f32),
        ],
        scratch_shapes=[
            pltpu.VMEM((B, SUB, LN), f32),
            pltpu.VMEM((B, 3, 1), f32),
        ],
    )(truths, truths_T, priors_R, loc_R, conf_R)

    return loss_l[0, 0], loss_c[0, 0]
